# baseline (speedup 1.0000x reference)
import functools

import jax
import jax.numpy as jnp
from jax.experimental import pallas as pl
from jax.experimental.pallas import tpu as pltpu

_K_BLK = 2048


def _mm_body(x_ref, w_ref, o_ref, *, k_total, k_blk, nk):
    k = pl.program_id(0)

    @pl.when(k == 0)
    def _init():
        o_ref[...] = jnp.zeros_like(o_ref)

    @pl.when(k < nk - 1)
    def _full():
        o_ref[...] += jax.lax.dot_general(
            x_ref[...], w_ref[...], (((1,), (0,)), ((), ())),
            preferred_element_type=jnp.float32,
        )

    @pl.when(k == nk - 1)
    def _tail():
        rem = k_total - (nk - 1) * k_blk
        xb = x_ref[...]
        wb = w_ref[...]
        col = jax.lax.broadcasted_iota(jnp.int32, xb.shape, 1)
        xb = jnp.where(col < rem, xb, 0.0)
        row = jax.lax.broadcasted_iota(jnp.int32, wb.shape, 0)
        wb = jnp.where(row < rem, wb, 0.0)
        o_ref[...] += jax.lax.dot_general(
            xb, wb, (((1,), (0,)), ((), ())),
            preferred_element_type=jnp.float32,
        )


def kernel(x, W):
    m, k_total = x.shape
    _, n = W.shape
    nk = pl.cdiv(k_total, _K_BLK)
    return pl.pallas_call(
        functools.partial(_mm_body, k_total=k_total, k_blk=_K_BLK, nk=nk),
        grid=(nk,),
        in_specs=[
            pl.BlockSpec((m, _K_BLK), lambda k: (0, k)),
            pl.BlockSpec((_K_BLK, n), lambda k: (k, 0)),
        ],
        out_specs=pl.BlockSpec((m, n), lambda k: (0, 0)),
        out_shape=jax.ShapeDtypeStruct((m, n), jnp.float32),
        compiler_params=pltpu.CompilerParams(
            dimension_semantics=("arbitrary",),
        ),
    )(x, W)


# speedup vs baseline: 1.0125x; 1.0125x over previous
import functools

import jax
import jax.numpy as jnp
from jax.experimental import pallas as pl
from jax.experimental.pallas import tpu as pltpu

_K_BLK = 2048


def _mm_body(x_ref, w_ref, o_ref, *, k_total, k_blk, nk):
    k = pl.program_id(0)

    @pl.when(k == 0)
    def _init():
        o_ref[...] = jnp.zeros_like(o_ref)

    @pl.when(k < nk - 1)
    def _full():
        o_ref[...] += jax.lax.dot_general(
            x_ref[...], w_ref[...], (((1,), (0,)), ((), ())),
            preferred_element_type=jnp.float32,
            precision=jax.lax.Precision.DEFAULT,
        )

    @pl.when(k == nk - 1)
    def _tail():
        rem = k_total - (nk - 1) * k_blk
        xb = x_ref[...]
        wb = w_ref[...]
        col = jax.lax.broadcasted_iota(jnp.int32, xb.shape, 1)
        xb = jnp.where(col < rem, xb, 0.0)
        row = jax.lax.broadcasted_iota(jnp.int32, wb.shape, 0)
        wb = jnp.where(row < rem, wb, 0.0)
        o_ref[...] += jax.lax.dot_general(
            xb, wb, (((1,), (0,)), ((), ())),
            preferred_element_type=jnp.float32,
            precision=jax.lax.Precision.DEFAULT,
        )


def kernel(x, W):
    m, k_total = x.shape
    _, n = W.shape
    nk = pl.cdiv(k_total, _K_BLK)
    return pl.pallas_call(
        functools.partial(_mm_body, k_total=k_total, k_blk=_K_BLK, nk=nk),
        grid=(nk,),
        in_specs=[
            pl.BlockSpec((m, _K_BLK), lambda k: (0, k)),
            pl.BlockSpec((_K_BLK, n), lambda k: (k, 0)),
        ],
        out_specs=pl.BlockSpec((m, n), lambda k: (0, 0)),
        out_shape=jax.ShapeDtypeStruct((m, n), jnp.float32),
        compiler_params=pltpu.CompilerParams(
            dimension_semantics=("arbitrary",),
        ),
    )(x, W)
